# baseline (speedup 1.0000x reference)
import functools

import jax
import jax.numpy as jnp
from jax import lax
from jax.experimental import pallas as pl
from jax.experimental.pallas import tpu as pltpu
from jax.experimental.pallas import tpu_sc as plsc

_NUM_EMB = 1001
_EMB_DIM = 64
_BATCH = 16384

_INFO = plsc.get_sparse_core_info()
_NC = _INFO.num_cores
_NS = _INFO.num_subcores
_NW = _NC * _NS
_B_PER_W = _BATCH // _NW
_IDX_MINOR = 128
_NCHUNK = _B_PER_W // _IDX_MINOR

_mesh = plsc.VectorSubcoreMesh(core_axis_name="c", subcore_axis_name="s")


@functools.partial(
    pl.kernel,
    mesh=_mesh,
    out_type=jax.ShapeDtypeStruct((_BATCH, _EMB_DIM), jnp.float32),
    scratch_types=[
        pltpu.VMEM((_NCHUNK, _IDX_MINOR), jnp.int32),
        pltpu.VMEM((_B_PER_W, _EMB_DIM), jnp.float32),
        pltpu.SemaphoreType.DMA((_NCHUNK,)),
        pltpu.SemaphoreType.DMA,
    ],
    compiler_params=pltpu.CompilerParams(use_tc_tiling_on_sc=False),
)
def _sc_gather(idx_hbm, table_hbm, out_hbm, idx_v, rows_v, gsem, wsem):
    wid = lax.axis_index("s") * _NC + lax.axis_index("c")
    base = wid * _B_PER_W
    pltpu.sync_copy(idx_hbm.at[pl.ds(wid * _NCHUNK, _NCHUNK)], idx_v)
    gathers = [
        pltpu.async_copy(
            table_hbm.at[idx_v.at[j]],
            rows_v.at[pl.ds(j * _IDX_MINOR, _IDX_MINOR)],
            gsem.at[j],
        )
        for j in range(_NCHUNK)
    ]
    writes = []
    for j, g in enumerate(gathers):
        g.wait()
        writes.append(
            pltpu.async_copy(
                rows_v.at[pl.ds(j * _IDX_MINOR, _IDX_MINOR)],
                out_hbm.at[pl.ds(base + j * _IDX_MINOR, _IDX_MINOR)],
                wsem,
            )
        )
    for w in writes:
        w.wait()


def kernel(user_ids, table):
    idx2d = user_ids.reshape(_NW * _NCHUNK, _IDX_MINOR)
    return _sc_gather(idx2d, table)


# speedup vs baseline: 1.2395x; 1.2395x over previous
import functools

import jax
import jax.numpy as jnp
from jax import lax
from jax.experimental import pallas as pl
from jax.experimental.pallas import tpu as pltpu
from jax.experimental.pallas import tpu_sc as plsc

_NUM_EMB = 1001
_EMB_DIM = 64
_BATCH = 16384

_INFO = plsc.get_sparse_core_info()
_NC = _INFO.num_cores
_NS = _INFO.num_subcores
_NW = _NC * _NS
_L = _INFO.num_lanes

_NDIMG = 8
_NBATG = _NW // _NDIMG
_ROWS = _EMB_DIM // _NDIMG
_BCOLS = _BATCH // _NBATG
_TPAD = 1008
_NTILES = _BCOLS // 128

_mesh = plsc.VectorSubcoreMesh(core_axis_name="c", subcore_axis_name="s")


@functools.partial(
    pl.kernel,
    mesh=_mesh,
    out_type=jax.ShapeDtypeStruct((_EMB_DIM, _BATCH), jnp.float32),
    scratch_types=[
        pltpu.VMEM((_ROWS * _TPAD,), jnp.float32),
        pltpu.VMEM((_BCOLS,), jnp.int32),
        pltpu.VMEM((_ROWS, 128), jnp.float32),
        pltpu.VMEM((_ROWS, 128), jnp.float32),
        pltpu.SemaphoreType.DMA,
        pltpu.SemaphoreType.DMA,
    ],
    compiler_params=pltpu.CompilerParams(
        use_tc_tiling_on_sc=True, needs_layout_passes=False
    ),
)
def _sc_gather_t(idx_hbm, tflat_hbm, out_hbm, tv, iv, tile_a, tile_b, sem_a, sem_b):
    wid = lax.axis_index("s") * _NC + lax.axis_index("c")
    g = wid % _NDIMG
    b = wid // _NDIMG
    pltpu.sync_copy(tflat_hbm.at[pl.ds(g * _ROWS * _TPAD, _ROWS * _TPAD)], tv)
    pltpu.sync_copy(idx_hbm.at[pl.ds(b * _BCOLS, _BCOLS)], iv)

    def build(tile, t):
        for c in range(128 // _L):
            ivec = iv[pl.ds(t * 128 + c * _L, _L)]
            for d in range(_ROWS):
                vec = plsc.load_gather(tv, [ivec + d * _TPAD])
                tile[d, pl.ds(c * _L, _L)] = vec

    def out_slice(t):
        return out_hbm.at[pl.ds(g * _ROWS, _ROWS), pl.ds(b * _BCOLS + t * 128, 128)]

    build(tile_a, 0)

    def body(i, carry):
        t0 = 2 * i
        wa = pltpu.async_copy(tile_a, out_slice(t0), sem_a)
        build(tile_b, t0 + 1)
        wa.wait()
        wb = pltpu.async_copy(tile_b, out_slice(t0 + 1), sem_b)
        build(tile_a, (t0 + 2) % _NTILES)
        wb.wait()
        return carry

    lax.fori_loop(0, _NTILES // 2, body, jnp.int32(0))


def kernel(user_ids, table):
    tflat = jnp.pad(table.T, ((0, 0), (0, _TPAD - _NUM_EMB))).reshape(-1)
    out_t = _sc_gather_t(user_ids, tflat)
    return jnp.transpose(out_t)


# speedup vs baseline: 1.5229x; 1.2286x over previous
import functools

import jax
import jax.numpy as jnp
from jax import lax
from jax.experimental import pallas as pl
from jax.experimental.pallas import tpu as pltpu
from jax.experimental.pallas import tpu_sc as plsc

_NUM_EMB = 1001
_EMB_DIM = 64
_BATCH = 16384

_INFO = plsc.get_sparse_core_info()
_NC = _INFO.num_cores
_NS = _INFO.num_subcores
_NW = _NC * _NS
_L = _INFO.num_lanes

_NDIMG = 8
_NBATG = _NW // _NDIMG
_ROWS = _EMB_DIM // _NDIMG
_BCOLS = _BATCH // _NBATG
_TPAD = 1008
_NTILES = _BCOLS // 128

_mesh = plsc.VectorSubcoreMesh(core_axis_name="c", subcore_axis_name="s")


@functools.partial(
    pl.kernel,
    mesh=_mesh,
    out_type=jax.ShapeDtypeStruct((_EMB_DIM, _BATCH), jnp.float32),
    scratch_types=[
        pltpu.VMEM((_ROWS * _TPAD,), jnp.float32),
        pltpu.VMEM((_BCOLS,), jnp.int32),
        pltpu.VMEM((_ROWS, 128), jnp.float32),
        pltpu.VMEM((_ROWS, 128), jnp.float32),
        pltpu.SemaphoreType.DMA,
        pltpu.SemaphoreType.DMA,
    ],
    compiler_params=pltpu.CompilerParams(
        use_tc_tiling_on_sc=True, needs_layout_passes=False
    ),
)
def _sc_gather_t(idx_hbm, tflat_hbm, out_hbm, tv, iv, tile_a, tile_b, sem_a, sem_b):
    wid = lax.axis_index("s") * _NC + lax.axis_index("c")
    g = wid % _NDIMG
    b = wid // _NDIMG
    pltpu.sync_copy(tflat_hbm.at[pl.ds(g * _ROWS * _TPAD, _ROWS * _TPAD)], tv)
    pltpu.sync_copy(idx_hbm.at[pl.ds(b * _BCOLS, _BCOLS)], iv)

    def build(tile, t):
        for c in range(128 // _L):
            ivec = iv[pl.ds(t * 128 + c * _L, _L)]
            addrs = [ivec + d * _TPAD for d in range(_ROWS)]
            vals = [plsc.load_gather(tv, [a]) for a in addrs]
            for d in range(_ROWS):
                tile[d, pl.ds(c * _L, _L)] = vals[d]

    def out_slice(t):
        return out_hbm.at[pl.ds(g * _ROWS, _ROWS), pl.ds(b * _BCOLS + t * 128, 128)]

    build(tile_a, 0)

    def body(i, carry):
        t0 = 2 * i
        wa = pltpu.async_copy(tile_a, out_slice(t0), sem_a)
        build(tile_b, t0 + 1)
        wa.wait()
        wb = pltpu.async_copy(tile_b, out_slice(t0 + 1), sem_b)
        build(tile_a, (t0 + 2) % _NTILES)
        wb.wait()
        return carry

    lax.fori_loop(0, _NTILES // 2, body, jnp.int32(0))


def kernel(user_ids, table):
    tflat = jnp.pad(table.T, ((0, 0), (0, _TPAD - _NUM_EMB))).reshape(-1)
    out_t = _sc_gather_t(user_ids, tflat)
    return jnp.transpose(out_t)
